# baseline (speedup 1.0000x reference)
import jax
import jax.numpy as jnp
from jax.experimental import pallas as pl
from jax.experimental.pallas import tpu as pltpu

EMBED_DIM = 32
NUM_EMB = 8192
COMMIT = 0.25
MOMENTUM = 0.9

N_POINTS = 8192
BLOCK = 256
GRID = N_POINTS // BLOCK


def _vq_kernel(x_ref, x2_ref, e_ref, cs_ref, un_ref,
               qst_ref, loss_ref, perp_ref, new_e_ref, new_cs_ref, new_un_ref,
               counts_acc, un_acc, loss_acc):
    i = pl.program_id(0)

    @pl.when(i == 0)
    def _init():
        counts_acc[...] = jnp.zeros_like(counts_acc)
        un_acc[...] = jnp.zeros_like(un_acc)
        loss_acc[...] = jnp.zeros_like(loss_acc)

    x_blk = x_ref[...]
    e = e_ref[...]

    x2 = x2_ref[...]
    e2 = jnp.sum(e * e, axis=0, keepdims=True)
    xb = x_blk.astype(jnp.bfloat16)
    eb = e.astype(jnp.bfloat16)
    m = jax.lax.dot_general(xb, eb, (((1,), (0,)), ((), ())),
                            preferred_element_type=jnp.float32)
    dist = (x2 + e2) - (m + m)

    HALF = NUM_EMB // 2
    d0 = dist[:, :HALF]
    d1 = dist[:, HALF:]
    md0 = jnp.min(d0, axis=1, keepdims=True)
    md1 = jnp.min(d1, axis=1, keepdims=True)
    hlane = jax.lax.broadcasted_iota(jnp.int32, d0.shape, 1)
    idx0 = jnp.min(jnp.where(d0 == md0, hlane, HALF), axis=1, keepdims=True)
    idx1 = jnp.min(jnp.where(d1 == md1, hlane, HALF), axis=1, keepdims=True)
    u0 = md0.astype(jnp.bfloat16).astype(jnp.float32)
    idx = jnp.where(u0 <= md1, idx0, idx1 + HALF)
    lane = jax.lax.broadcasted_iota(jnp.int32, dist.shape, 1)

    one_hot = (lane == idx).astype(jnp.float32)

    q_blk = jax.lax.dot_general(one_hot, e, (((1,), (1,)), ((), ())),
                                preferred_element_type=jnp.float32, precision=jax.lax.Precision.HIGHEST)
    qst_ref[...] = x_blk + (q_blk - x_blk)

    diff = q_blk - x_blk
    loss_acc[...] += jnp.sum(diff * diff).reshape(1, 1)

    counts_acc[...] += jnp.sum(one_hot, axis=0, keepdims=True)
    un_acc[...] += jax.lax.dot_general(xb, one_hot.astype(jnp.bfloat16),
                                       (((0,), (0,)), ((), ())),
                                       preferred_element_type=jnp.float32)

    @pl.when(i == GRID - 1)
    def _finalize():
        counts = counts_acc[...]
        new_cs = (1 - MOMENTUM) * counts + MOMENTUM * cs_ref[...]
        new_un = (1 - MOMENTUM) * un_acc[...] + MOMENTUM * un_ref[...]
        n = jnp.sum(new_cs)
        stable_cs = (new_cs + 1e-20) / (n + NUM_EMB * 1e-20) * n
        new_cs_ref[...] = new_cs
        new_un_ref[...] = new_un
        new_e_ref[...] = new_un / stable_cs
        loss_ref[...] = COMMIT * (loss_acc[...] / (N_POINTS * EMBED_DIM))
        ol = counts / N_POINTS
        perp_ref[...] = jnp.exp(-jnp.sum(ol * jnp.log(ol + 1e-20))).reshape(1, 1)


@jax.jit
def kernel(x, e, cs, un):
    xf = x.reshape(N_POINTS, EMBED_DIM)
    x2 = jnp.sum(xf * xf, 1, keepdims=True)
    cs2 = cs.reshape(1, NUM_EMB)

    out_shapes = (
        jax.ShapeDtypeStruct((N_POINTS, EMBED_DIM), jnp.float32),
        jax.ShapeDtypeStruct((1, 1), jnp.float32),
        jax.ShapeDtypeStruct((1, 1), jnp.float32),
        jax.ShapeDtypeStruct((EMBED_DIM, NUM_EMB), jnp.float32),
        jax.ShapeDtypeStruct((1, NUM_EMB), jnp.float32),
        jax.ShapeDtypeStruct((EMBED_DIM, NUM_EMB), jnp.float32),
    )

    grid = (GRID,)
    in_specs = [
            pl.BlockSpec((BLOCK, EMBED_DIM), lambda i: (i, 0)),
            pl.BlockSpec((BLOCK, 1), lambda i: (i, 0)),
            pl.BlockSpec((EMBED_DIM, NUM_EMB), lambda i: (0, 0)),
            pl.BlockSpec((1, NUM_EMB), lambda i: (0, 0)),
            pl.BlockSpec((EMBED_DIM, NUM_EMB), lambda i: (0, 0)),
    ]
    out_specs = (
            pl.BlockSpec((BLOCK, EMBED_DIM), lambda i: (i, 0)),
            pl.BlockSpec((1, 1), lambda i: (0, 0)),
            pl.BlockSpec((1, 1), lambda i: (0, 0)),
            pl.BlockSpec((EMBED_DIM, NUM_EMB), lambda i: (0, 0)),
            pl.BlockSpec((1, NUM_EMB), lambda i: (0, 0)),
            pl.BlockSpec((EMBED_DIM, NUM_EMB), lambda i: (0, 0)),
    )

    qst, loss, perp, new_e, new_cs, new_un = pl.pallas_call(
        _vq_kernel,
        grid=grid,
        in_specs=in_specs,
        out_specs=out_specs,
        out_shape=out_shapes,
        scratch_shapes=[
            pltpu.VMEM((1, NUM_EMB), jnp.float32),
            pltpu.VMEM((EMBED_DIM, NUM_EMB), jnp.float32),
            pltpu.VMEM((1, 1), jnp.float32),
        ],
    )(xf, x2, e, cs2, un)

    return (qst.reshape(x.shape), loss[0, 0], perp[0, 0],
            new_e, new_cs.reshape(NUM_EMB), new_un)


# speedup vs baseline: 1.5602x; 1.5602x over previous
import jax
import jax.numpy as jnp
from jax.experimental import pallas as pl
from jax.experimental.pallas import tpu as pltpu

EMBED_DIM = 32
NUM_EMB = 8192
COMMIT = 0.25
MOMENTUM = 0.9

N_POINTS = 8192
BLOCK = 256
GRID = N_POINTS // BLOCK


def _vq_kernel(x_ref, x2_ref, e_ref, cs_ref, un_ref,
               qst_ref, loss_ref, perp_ref, new_e_ref, new_cs_ref, new_un_ref,
               counts_acc, un_acc, loss_acc):
    i = pl.program_id(0)

    @pl.when(i == 0)
    def _init():
        counts_acc[...] = jnp.zeros_like(counts_acc)
        un_acc[...] = jnp.zeros_like(un_acc)
        loss_acc[...] = jnp.zeros_like(loss_acc)

    x_blk = x_ref[...]
    e = e_ref[...]

    x2 = x2_ref[...]
    e2 = jnp.sum(e * e, axis=0, keepdims=True)
    xb = x_blk.astype(jnp.bfloat16)
    eb = e.astype(jnp.bfloat16)
    m = jax.lax.dot_general(xb, eb, (((1,), (0,)), ((), ())),
                            preferred_element_type=jnp.float32)
    dist = (x2 + e2) - (m + m)

    HALF = NUM_EMB // 2
    d0 = dist[:, :HALF]
    d1 = dist[:, HALF:]
    md0 = jnp.min(d0, axis=1, keepdims=True)
    md1 = jnp.min(d1, axis=1, keepdims=True)
    hlane = jax.lax.broadcasted_iota(jnp.int32, d0.shape, 1)
    idx0 = jnp.min(jnp.where(d0 == md0, hlane, HALF), axis=1, keepdims=True)
    idx1 = jnp.min(jnp.where(d1 == md1, hlane, HALF), axis=1, keepdims=True)
    u0 = md0.astype(jnp.bfloat16).astype(jnp.float32)
    idx = jnp.where(u0 <= md1, idx0, idx1 + HALF)
    lane = jax.lax.broadcasted_iota(jnp.int32, dist.shape, 1)

    one_hot = (lane == idx).astype(jnp.float32)

    oh_b = one_hot.astype(jnp.bfloat16)
    e_lo = (e - eb.astype(jnp.float32)).astype(jnp.bfloat16)
    q_blk = (jax.lax.dot_general(oh_b, eb, (((1,), (1,)), ((), ())),
                                 preferred_element_type=jnp.float32)
             + jax.lax.dot_general(oh_b, e_lo, (((1,), (1,)), ((), ())),
                                   preferred_element_type=jnp.float32))
    qst_ref[...] = x_blk + (q_blk - x_blk)

    mdw = jnp.where(u0 <= md1, md0, md1)
    loss_acc[...] += jnp.sum(mdw).reshape(1, 1)

    counts_acc[...] += jnp.sum(one_hot, axis=0, keepdims=True)
    un_acc[...] += jax.lax.dot_general(xb, oh_b, (((0,), (0,)), ((), ())),
                                       preferred_element_type=jnp.float32)

    @pl.when(i == GRID - 1)
    def _finalize():
        counts = counts_acc[...]
        new_cs = (1 - MOMENTUM) * counts + MOMENTUM * cs_ref[...]
        new_un = (1 - MOMENTUM) * un_acc[...] + MOMENTUM * un_ref[...]
        n = jnp.sum(new_cs)
        stable_cs = (new_cs + 1e-20) / (n + NUM_EMB * 1e-20) * n
        new_cs_ref[...] = new_cs
        new_un_ref[...] = new_un
        new_e_ref[...] = new_un / stable_cs
        loss_ref[...] = COMMIT * (loss_acc[...] / (N_POINTS * EMBED_DIM))
        ol = counts / N_POINTS
        perp_ref[...] = jnp.exp(-jnp.sum(ol * jnp.log(ol + 1e-20))).reshape(1, 1)


@jax.jit
def kernel(x, e, cs, un):
    xf = x.reshape(N_POINTS, EMBED_DIM)
    x2 = jnp.sum(xf * xf, 1, keepdims=True)
    cs2 = cs.reshape(1, NUM_EMB)

    out_shapes = (
        jax.ShapeDtypeStruct((N_POINTS, EMBED_DIM), jnp.float32),
        jax.ShapeDtypeStruct((1, 1), jnp.float32),
        jax.ShapeDtypeStruct((1, 1), jnp.float32),
        jax.ShapeDtypeStruct((EMBED_DIM, NUM_EMB), jnp.float32),
        jax.ShapeDtypeStruct((1, NUM_EMB), jnp.float32),
        jax.ShapeDtypeStruct((EMBED_DIM, NUM_EMB), jnp.float32),
    )

    grid = (GRID,)
    in_specs = [
            pl.BlockSpec((BLOCK, EMBED_DIM), lambda i: (i, 0)),
            pl.BlockSpec((BLOCK, 1), lambda i: (i, 0)),
            pl.BlockSpec((EMBED_DIM, NUM_EMB), lambda i: (0, 0)),
            pl.BlockSpec((1, NUM_EMB), lambda i: (0, 0)),
            pl.BlockSpec((EMBED_DIM, NUM_EMB), lambda i: (0, 0)),
    ]
    out_specs = (
            pl.BlockSpec((BLOCK, EMBED_DIM), lambda i: (i, 0)),
            pl.BlockSpec((1, 1), lambda i: (0, 0)),
            pl.BlockSpec((1, 1), lambda i: (0, 0)),
            pl.BlockSpec((EMBED_DIM, NUM_EMB), lambda i: (0, 0)),
            pl.BlockSpec((1, NUM_EMB), lambda i: (0, 0)),
            pl.BlockSpec((EMBED_DIM, NUM_EMB), lambda i: (0, 0)),
    )

    qst, loss, perp, new_e, new_cs, new_un = pl.pallas_call(
        _vq_kernel,
        grid=grid,
        in_specs=in_specs,
        out_specs=out_specs,
        out_shape=out_shapes,
        scratch_shapes=[
            pltpu.VMEM((1, NUM_EMB), jnp.float32),
            pltpu.VMEM((EMBED_DIM, NUM_EMB), jnp.float32),
            pltpu.VMEM((1, 1), jnp.float32),
        ],
    )(xf, x2, e, cs2, un)

    return (qst.reshape(x.shape), loss[0, 0], perp[0, 0],
            new_e, new_cs.reshape(NUM_EMB), new_un)


# speedup vs baseline: 1.6181x; 1.0371x over previous
import jax
import jax.numpy as jnp
from jax.experimental import pallas as pl
from jax.experimental.pallas import tpu as pltpu

EMBED_DIM = 32
NUM_EMB = 8192
COMMIT = 0.25
MOMENTUM = 0.9

N_POINTS = 8192
BLOCK = 256
GRID = N_POINTS // BLOCK


def _vq_kernel(x_ref, x2_ref, e_ref, cs_ref, un_ref,
               qst_ref, loss_ref, perp_ref, new_e_ref, new_cs_ref, new_un_ref,
               un_acc, loss_acc, e2_acc):
    i = pl.program_id(0)

    x_blk = x_ref[...]
    e = e_ref[...]

    @pl.when(i == 0)
    def _init():
        un_acc[...] = jnp.zeros_like(un_acc)
        loss_acc[...] = jnp.zeros_like(loss_acc)
        e2_acc[...] = jnp.sum(e * e, axis=0, keepdims=True)

    x2 = x2_ref[...]
    e2 = e2_acc[...]
    xb = x_blk.astype(jnp.bfloat16)
    eb = e.astype(jnp.bfloat16)
    m2 = jax.lax.dot_general((x_blk + x_blk).astype(jnp.bfloat16), eb,
                             (((1,), (0,)), ((), ())),
                             preferred_element_type=jnp.float32)
    dist = (x2 + e2) - m2

    HALF = NUM_EMB // 2
    d0 = dist[:, :HALF]
    d1 = dist[:, HALF:]
    md0 = jnp.min(d0, axis=1, keepdims=True)
    md1 = jnp.min(d1, axis=1, keepdims=True)
    hlane = jax.lax.broadcasted_iota(jnp.int32, d0.shape, 1)
    idx0 = jnp.min(jnp.where(d0 == md0, hlane, HALF), axis=1, keepdims=True)
    idx1 = jnp.min(jnp.where(d1 == md1, hlane, HALF), axis=1, keepdims=True)
    u0 = md0.astype(jnp.bfloat16).astype(jnp.float32)
    idx = jnp.where(u0 <= md1, idx0, idx1 + HALF)
    lane = jax.lax.broadcasted_iota(jnp.int32, dist.shape, 1)

    oh_b = (lane == idx).astype(jnp.bfloat16)

    e_lo = (e - eb.astype(jnp.float32)).astype(jnp.bfloat16)
    q_blk = (jax.lax.dot_general(oh_b, eb, (((1,), (1,)), ((), ())),
                                 preferred_element_type=jnp.float32)
             + jax.lax.dot_general(oh_b, e_lo, (((1,), (1,)), ((), ())),
                                   preferred_element_type=jnp.float32))
    qst_ref[...] = x_blk + (q_blk - x_blk)

    mdw = jnp.where(u0 <= md1, md0, md1)
    loss_acc[...] += jnp.sum(mdw).reshape(1, 1)

    xb_aug = jnp.concatenate(
        [xb, jnp.ones((BLOCK, 1), jnp.bfloat16)], axis=1)
    un_acc[...] += jax.lax.dot_general(xb_aug, oh_b, (((0,), (0,)), ((), ())),
                                       preferred_element_type=jnp.float32)

    @pl.when(i == GRID - 1)
    def _finalize():
        counts = un_acc[EMBED_DIM:, :]
        new_cs = (1 - MOMENTUM) * counts + MOMENTUM * cs_ref[...]
        new_un = (1 - MOMENTUM) * un_acc[:EMBED_DIM, :] + MOMENTUM * un_ref[...]
        n = jnp.sum(new_cs)
        stable_cs = (new_cs + 1e-20) / (n + NUM_EMB * 1e-20) * n
        new_cs_ref[...] = new_cs
        new_un_ref[...] = new_un
        new_e_ref[...] = new_un / stable_cs
        loss_ref[...] = COMMIT * (loss_acc[...] / (N_POINTS * EMBED_DIM))
        ol = counts / N_POINTS
        perp_ref[...] = jnp.exp(-jnp.sum(ol * jnp.log(ol + 1e-20))).reshape(1, 1)


@jax.jit
def kernel(x, e, cs, un):
    xf = x.reshape(N_POINTS, EMBED_DIM)
    x2 = jnp.sum(xf * xf, 1, keepdims=True)
    cs2 = cs.reshape(1, NUM_EMB)

    out_shapes = (
        jax.ShapeDtypeStruct((N_POINTS, EMBED_DIM), jnp.float32),
        jax.ShapeDtypeStruct((1, 1), jnp.float32),
        jax.ShapeDtypeStruct((1, 1), jnp.float32),
        jax.ShapeDtypeStruct((EMBED_DIM, NUM_EMB), jnp.float32),
        jax.ShapeDtypeStruct((1, NUM_EMB), jnp.float32),
        jax.ShapeDtypeStruct((EMBED_DIM, NUM_EMB), jnp.float32),
    )

    grid = (GRID,)
    in_specs = [
            pl.BlockSpec((BLOCK, EMBED_DIM), lambda i: (i, 0)),
            pl.BlockSpec((BLOCK, 1), lambda i: (i, 0)),
            pl.BlockSpec((EMBED_DIM, NUM_EMB), lambda i: (0, 0)),
            pl.BlockSpec((1, NUM_EMB), lambda i: (0, 0)),
            pl.BlockSpec((EMBED_DIM, NUM_EMB), lambda i: (0, 0)),
    ]
    out_specs = (
            pl.BlockSpec((BLOCK, EMBED_DIM), lambda i: (i, 0)),
            pl.BlockSpec((1, 1), lambda i: (0, 0)),
            pl.BlockSpec((1, 1), lambda i: (0, 0)),
            pl.BlockSpec((EMBED_DIM, NUM_EMB), lambda i: (0, 0)),
            pl.BlockSpec((1, NUM_EMB), lambda i: (0, 0)),
            pl.BlockSpec((EMBED_DIM, NUM_EMB), lambda i: (0, 0)),
    )

    qst, loss, perp, new_e, new_cs, new_un = pl.pallas_call(
        _vq_kernel,
        grid=grid,
        in_specs=in_specs,
        out_specs=out_specs,
        out_shape=out_shapes,
        scratch_shapes=[
            pltpu.VMEM((EMBED_DIM + 1, NUM_EMB), jnp.float32),
            pltpu.VMEM((1, 1), jnp.float32),
            pltpu.VMEM((1, NUM_EMB), jnp.float32),
        ],
    )(xf, x2, e, cs2, un)

    return (qst.reshape(x.shape), loss[0, 0], perp[0, 0],
            new_e, new_cs.reshape(NUM_EMB), new_un)


# speedup vs baseline: 2.3837x; 1.4732x over previous
import jax
import jax.numpy as jnp
from jax import lax
from jax.experimental import pallas as pl
from jax.experimental.pallas import tpu as pltpu
from jax.experimental.pallas import tpu_sc as plsc

EMBED_DIM = 32
NUM_EMB = 8192
COMMIT = 0.25
MOMENTUM = 0.9

N_POINTS = 8192
BLOCK = 256
GRID = N_POINTS // BLOCK
GAUG = 128

NC = 2
NS = 16
NW = NC * NS
BPW = N_POINTS // NW


def _vq_kernel(x_ref, x2_ref, e_ref, cs_ref, un_ref,
               idx_ref, et_ref, loss_ref, perp_ref, new_e_ref, new_cs_ref,
               new_un_ref,
               un_acc, loss_acc, e2_acc):
    i = pl.program_id(0)

    x_blk = x_ref[...]
    e = e_ref[...]
    eb = e.astype(jnp.bfloat16)

    @pl.when(i == 0)
    def _init():
        un_acc[...] = jnp.zeros_like(un_acc)
        loss_acc[...] = jnp.zeros_like(loss_acc)
        e2_acc[...] = jnp.sum(e * e, axis=0, keepdims=True)
        e_lo = (e - eb.astype(jnp.float32)).astype(jnp.bfloat16)
        eye = (jax.lax.broadcasted_iota(jnp.int32, (EMBED_DIM, GAUG), 0)
               == jax.lax.broadcasted_iota(jnp.int32, (EMBED_DIM, GAUG), 1)
               ).astype(jnp.bfloat16)
        et_ref[...] = (
            jax.lax.dot_general(eb, eye, (((0,), (0,)), ((), ())),
                                preferred_element_type=jnp.float32)
            + jax.lax.dot_general(e_lo, eye, (((0,), (0,)), ((), ())),
                                  preferred_element_type=jnp.float32))

    x2 = x2_ref[...]
    e2 = e2_acc[...]
    m2 = jax.lax.dot_general((x_blk + x_blk).astype(jnp.bfloat16), eb,
                             (((1,), (0,)), ((), ())),
                             preferred_element_type=jnp.float32)
    dist = (x2 + e2) - m2

    HALF = NUM_EMB // 2
    d0 = dist[:, :HALF]
    d1 = dist[:, HALF:]
    md0 = jnp.min(d0, axis=1, keepdims=True)
    md1 = jnp.min(d1, axis=1, keepdims=True)
    hlane = jax.lax.broadcasted_iota(jnp.int32, d0.shape, 1)
    idx0 = jnp.min(jnp.where(d0 == md0, hlane, HALF), axis=1, keepdims=True)
    idx1 = jnp.min(jnp.where(d1 == md1, hlane, HALF), axis=1, keepdims=True)
    u0 = md0.astype(jnp.bfloat16).astype(jnp.float32)
    pick0 = u0 <= md1
    idx = jnp.where(pick0, idx0, idx1 + HALF)
    idx_ref[...] = idx

    lane = jax.lax.broadcasted_iota(jnp.int32, dist.shape, 1)
    oh_b = (lane == idx).astype(jnp.bfloat16)

    xb_aug = jnp.concatenate(
        [x_blk.astype(jnp.bfloat16), jnp.ones((BLOCK, 1), jnp.bfloat16)],
        axis=1)
    un_acc[...] += jax.lax.dot_general(xb_aug, oh_b, (((0,), (0,)), ((), ())),
                                       preferred_element_type=jnp.float32)

    loss_acc[...] += jnp.sum(jnp.where(pick0, md0, md1)).reshape(1, 1)

    @pl.when(i == GRID - 1)
    def _finalize():
        counts = un_acc[EMBED_DIM:, :]
        new_cs = (1 - MOMENTUM) * counts + MOMENTUM * cs_ref[...]
        new_un = (1 - MOMENTUM) * un_acc[:EMBED_DIM, :] + MOMENTUM * un_ref[...]
        n = jnp.sum(new_cs)
        stable_cs = (new_cs + 1e-20) / (n + NUM_EMB * 1e-20) * n
        new_cs_ref[...] = new_cs
        new_un_ref[...] = new_un
        new_e_ref[...] = new_un / stable_cs
        loss_ref[...] = COMMIT * (loss_acc[...] / (N_POINTS * EMBED_DIM))
        ol = counts / N_POINTS
        perp_ref[...] = jnp.exp(-jnp.sum(ol * jnp.log(ol + 1e-20))).reshape(1, 1)


def _sc_gather(et_hbm, idx_hbm, q_hbm, idx_a, idx_b, q_a, q_b, sem):
    c = lax.axis_index("c")
    s = lax.axis_index("s")
    wid = s * NC + c
    base = wid * BPW

    pltpu.sync_copy(idx_hbm.at[pl.ds(base, 128)], idx_a)
    pltpu.sync_copy(idx_hbm.at[pl.ds(base + 128, 128)], idx_b)

    pltpu.async_copy(et_hbm.at[idx_a], q_a, sem).wait()
    pltpu.async_copy(et_hbm.at[idx_b], q_b, sem).wait()
    pltpu.sync_copy(q_a, q_hbm.at[pl.ds(base, 128)])
    pltpu.sync_copy(q_b, q_hbm.at[pl.ds(base + 128, 128)])


def _qst_kernel(q_ref, x_ref, qst_ref):
    x = x_ref[...]
    q = q_ref[...][:, :EMBED_DIM]
    qst_ref[...] = x + (q - x)


@jax.jit
def kernel(x, e, cs, un):
    xf = x.reshape(N_POINTS, EMBED_DIM)
    x2 = jnp.sum(xf * xf, 1, keepdims=True)
    cs2 = cs.reshape(1, NUM_EMB)

    idx, et, loss, perp, new_e, new_cs, new_un = pl.pallas_call(
        _vq_kernel,
        grid=(GRID,),
        in_specs=[
            pl.BlockSpec((BLOCK, EMBED_DIM), lambda i: (i, 0)),
            pl.BlockSpec((BLOCK, 1), lambda i: (i, 0)),
            pl.BlockSpec((EMBED_DIM, NUM_EMB), lambda i: (0, 0)),
            pl.BlockSpec((1, NUM_EMB), lambda i: (0, 0)),
            pl.BlockSpec((EMBED_DIM, NUM_EMB), lambda i: (0, 0)),
        ],
        out_specs=(
            pl.BlockSpec((BLOCK, 1), lambda i: (i, 0)),
            pl.BlockSpec((NUM_EMB, GAUG), lambda i: (0, 0)),
            pl.BlockSpec((1, 1), lambda i: (0, 0)),
            pl.BlockSpec((1, 1), lambda i: (0, 0)),
            pl.BlockSpec((EMBED_DIM, NUM_EMB), lambda i: (0, 0)),
            pl.BlockSpec((1, NUM_EMB), lambda i: (0, 0)),
            pl.BlockSpec((EMBED_DIM, NUM_EMB), lambda i: (0, 0)),
        ),
        out_shape=(
            jax.ShapeDtypeStruct((N_POINTS, 1), jnp.int32),
            jax.ShapeDtypeStruct((NUM_EMB, GAUG), jnp.float32),
            jax.ShapeDtypeStruct((1, 1), jnp.float32),
            jax.ShapeDtypeStruct((1, 1), jnp.float32),
            jax.ShapeDtypeStruct((EMBED_DIM, NUM_EMB), jnp.float32),
            jax.ShapeDtypeStruct((1, NUM_EMB), jnp.float32),
            jax.ShapeDtypeStruct((EMBED_DIM, NUM_EMB), jnp.float32),
        ),
        scratch_shapes=[
            pltpu.VMEM((EMBED_DIM + 1, NUM_EMB), jnp.float32),
            pltpu.VMEM((1, 1), jnp.float32),
            pltpu.VMEM((1, NUM_EMB), jnp.float32),
        ],
    )(xf, x2, e, cs2, un)

    mesh = plsc.VectorSubcoreMesh(core_axis_name="c", subcore_axis_name="s")
    q = pl.kernel(
        _sc_gather,
        out_type=jax.ShapeDtypeStruct((N_POINTS, GAUG), jnp.float32),
        mesh=mesh,
        scratch_types=[
            pltpu.VMEM((128,), jnp.int32),
            pltpu.VMEM((128,), jnp.int32),
            pltpu.VMEM((128, GAUG), jnp.float32),
            pltpu.VMEM((128, GAUG), jnp.float32),
            pltpu.SemaphoreType.DMA,
        ],
    )(et, idx.reshape(N_POINTS))

    qst = pl.pallas_call(
        _qst_kernel,
        out_shape=jax.ShapeDtypeStruct((N_POINTS, EMBED_DIM), jnp.float32),
    )(q, xf)

    return (qst.reshape(x.shape), loss[0, 0], perp[0, 0],
            new_e, new_cs.reshape(NUM_EMB), new_un)


# speedup vs baseline: 2.7622x; 1.1588x over previous
import jax
import jax.numpy as jnp
from jax import lax
from jax.experimental import pallas as pl
from jax.experimental.pallas import tpu as pltpu
from jax.experimental.pallas import tpu_sc as plsc

EMBED_DIM = 32
NUM_EMB = 8192
COMMIT = 0.25
MOMENTUM = 0.9

N_POINTS = 8192
BLOCK = 256
GRID = N_POINTS // BLOCK
GAUG = 128

NC = 2
NS = 16
NW = NC * NS
BPW = N_POINTS // NW


def _vq_kernel(x_ref, x2_ref, e_ref, cs_ref, un_ref,
               idx_ref, et_ref, loss_ref, perp_ref, new_e_ref, new_cs_ref,
               new_un_ref,
               un_acc, loss_acc, e2_acc):
    i = pl.program_id(0)

    x_blk = x_ref[...]
    e = e_ref[...]
    eb = e.astype(jnp.bfloat16)

    @pl.when(i == 0)
    def _init():
        un_acc[...] = jnp.zeros_like(un_acc)
        loss_acc[...] = jnp.zeros_like(loss_acc)
        e2_acc[...] = jnp.sum(e * e, axis=0, keepdims=True)
        e_lo = (e - eb.astype(jnp.float32)).astype(jnp.bfloat16)
        eye = (jax.lax.broadcasted_iota(jnp.int32, (EMBED_DIM, GAUG), 0)
               == jax.lax.broadcasted_iota(jnp.int32, (EMBED_DIM, GAUG), 1)
               ).astype(jnp.bfloat16)
        et_ref[...] = (
            jax.lax.dot_general(eb, eye, (((0,), (0,)), ((), ())),
                                preferred_element_type=jnp.float32)
            + jax.lax.dot_general(e_lo, eye, (((0,), (0,)), ((), ())),
                                  preferred_element_type=jnp.float32))

    x2 = x2_ref[...]
    e2 = e2_acc[...]
    m2 = jax.lax.dot_general((x_blk + x_blk).astype(jnp.bfloat16), eb,
                             (((1,), (0,)), ((), ())),
                             preferred_element_type=jnp.float32)
    dist = (x2 + e2) - m2

    HALF = NUM_EMB // 2
    d0 = dist[:, :HALF]
    d1 = dist[:, HALF:]
    md0 = jnp.min(d0, axis=1, keepdims=True)
    md1 = jnp.min(d1, axis=1, keepdims=True)
    idx0 = jnp.argmin(d0, axis=1, keepdims=True).astype(jnp.int32)
    idx1 = jnp.argmin(d1, axis=1, keepdims=True).astype(jnp.int32)
    u0 = md0.astype(jnp.bfloat16).astype(jnp.float32)
    pick0 = u0 <= md1
    idx = jnp.where(pick0, idx0, idx1 + HALF)
    idx_ref[...] = idx

    lane = jax.lax.broadcasted_iota(jnp.int32, dist.shape, 1)
    oh_b = (lane == idx).astype(jnp.bfloat16)

    xb_aug = jnp.concatenate(
        [x_blk.astype(jnp.bfloat16), jnp.ones((BLOCK, 1), jnp.bfloat16)],
        axis=1)
    un_acc[...] += jax.lax.dot_general(xb_aug, oh_b, (((0,), (0,)), ((), ())),
                                       preferred_element_type=jnp.float32)

    loss_acc[...] += jnp.sum(jnp.where(pick0, md0, md1)).reshape(1, 1)

    @pl.when(i == GRID - 1)
    def _finalize():
        counts = un_acc[EMBED_DIM:, :]
        new_cs = (1 - MOMENTUM) * counts + MOMENTUM * cs_ref[...]
        new_un = (1 - MOMENTUM) * un_acc[:EMBED_DIM, :] + MOMENTUM * un_ref[...]
        n = jnp.sum(new_cs)
        stable_cs = (new_cs + 1e-20) / (n + NUM_EMB * 1e-20) * n
        new_cs_ref[...] = new_cs
        new_un_ref[...] = new_un
        new_e_ref[...] = new_un / stable_cs
        loss_ref[...] = COMMIT * (loss_acc[...] / (N_POINTS * EMBED_DIM))
        ol = counts / N_POINTS
        perp_ref[...] = jnp.exp(-jnp.sum(ol * jnp.log(ol + 1e-20))).reshape(1, 1)


def _sc_gather(et_hbm, idx_hbm, q_hbm, idx_a, idx_b, q_a, q_b, sem):
    c = lax.axis_index("c")
    s = lax.axis_index("s")
    wid = s * NC + c
    base = wid * BPW

    pltpu.sync_copy(idx_hbm.at[pl.ds(base, 128)], idx_a)
    pltpu.sync_copy(idx_hbm.at[pl.ds(base + 128, 128)], idx_b)

    pltpu.async_copy(et_hbm.at[idx_a], q_a, sem).wait()
    pltpu.async_copy(et_hbm.at[idx_b], q_b, sem).wait()
    pltpu.sync_copy(q_a, q_hbm.at[pl.ds(base, 128)])
    pltpu.sync_copy(q_b, q_hbm.at[pl.ds(base + 128, 128)])


def _qst_kernel(q_ref, x_ref, qst_ref):
    x = x_ref[...]
    q = q_ref[...][:, :EMBED_DIM]
    qst_ref[...] = x + (q - x)


@jax.jit
def kernel(x, e, cs, un):
    xf = x.reshape(N_POINTS, EMBED_DIM)
    x2 = jnp.sum(xf * xf, 1, keepdims=True)
    cs2 = cs.reshape(1, NUM_EMB)

    idx, et, loss, perp, new_e, new_cs, new_un = pl.pallas_call(
        _vq_kernel,
        grid=(GRID,),
        in_specs=[
            pl.BlockSpec((BLOCK, EMBED_DIM), lambda i: (i, 0)),
            pl.BlockSpec((BLOCK, 1), lambda i: (i, 0)),
            pl.BlockSpec((EMBED_DIM, NUM_EMB), lambda i: (0, 0)),
            pl.BlockSpec((1, NUM_EMB), lambda i: (0, 0)),
            pl.BlockSpec((EMBED_DIM, NUM_EMB), lambda i: (0, 0)),
        ],
        out_specs=(
            pl.BlockSpec((BLOCK, 1), lambda i: (i, 0)),
            pl.BlockSpec((NUM_EMB, GAUG), lambda i: (0, 0)),
            pl.BlockSpec((1, 1), lambda i: (0, 0)),
            pl.BlockSpec((1, 1), lambda i: (0, 0)),
            pl.BlockSpec((EMBED_DIM, NUM_EMB), lambda i: (0, 0)),
            pl.BlockSpec((1, NUM_EMB), lambda i: (0, 0)),
            pl.BlockSpec((EMBED_DIM, NUM_EMB), lambda i: (0, 0)),
        ),
        out_shape=(
            jax.ShapeDtypeStruct((N_POINTS, 1), jnp.int32),
            jax.ShapeDtypeStruct((NUM_EMB, GAUG), jnp.float32),
            jax.ShapeDtypeStruct((1, 1), jnp.float32),
            jax.ShapeDtypeStruct((1, 1), jnp.float32),
            jax.ShapeDtypeStruct((EMBED_DIM, NUM_EMB), jnp.float32),
            jax.ShapeDtypeStruct((1, NUM_EMB), jnp.float32),
            jax.ShapeDtypeStruct((EMBED_DIM, NUM_EMB), jnp.float32),
        ),
        scratch_shapes=[
            pltpu.VMEM((EMBED_DIM + 1, NUM_EMB), jnp.float32),
            pltpu.VMEM((1, 1), jnp.float32),
            pltpu.VMEM((1, NUM_EMB), jnp.float32),
        ],
    )(xf, x2, e, cs2, un)

    mesh = plsc.VectorSubcoreMesh(core_axis_name="c", subcore_axis_name="s")
    q = pl.kernel(
        _sc_gather,
        out_type=jax.ShapeDtypeStruct((N_POINTS, GAUG), jnp.float32),
        mesh=mesh,
        scratch_types=[
            pltpu.VMEM((128,), jnp.int32),
            pltpu.VMEM((128,), jnp.int32),
            pltpu.VMEM((128, GAUG), jnp.float32),
            pltpu.VMEM((128, GAUG), jnp.float32),
            pltpu.SemaphoreType.DMA,
        ],
    )(et, idx.reshape(N_POINTS))

    qst = pl.pallas_call(
        _qst_kernel,
        out_shape=jax.ShapeDtypeStruct((N_POINTS, EMBED_DIM), jnp.float32),
    )(q, xf)

    return (qst.reshape(x.shape), loss[0, 0], perp[0, 0],
            new_e, new_cs.reshape(NUM_EMB), new_un)


# speedup vs baseline: 2.8388x; 1.0277x over previous
import jax
import jax.numpy as jnp
from jax import lax
from jax.experimental import pallas as pl
from jax.experimental.pallas import tpu as pltpu
from jax.experimental.pallas import tpu_sc as plsc

EMBED_DIM = 32
NUM_EMB = 8192
COMMIT = 0.25
MOMENTUM = 0.9

N_POINTS = 8192
BLOCK = 512
GRID = N_POINTS // BLOCK
GAUG = 128

NC = 2
NS = 16
NW = NC * NS
BPW = N_POINTS // NW


def _vq_kernel(x_ref, x2_ref, e_ref, cs_ref, un_ref,
               idx_ref, et_ref, loss_ref, perp_ref, new_e_ref, new_cs_ref,
               new_un_ref,
               un_acc, loss_acc, e2_acc):
    i = pl.program_id(0)

    x_blk = x_ref[...]
    e = e_ref[...]
    eb = e.astype(jnp.bfloat16)

    @pl.when(i == 0)
    def _init():
        un_acc[...] = jnp.zeros_like(un_acc)
        loss_acc[...] = jnp.zeros_like(loss_acc)
        e2_acc[...] = jnp.sum(e * e, axis=0, keepdims=True)
        e_lo = (e - eb.astype(jnp.float32)).astype(jnp.bfloat16)
        eye = (jax.lax.broadcasted_iota(jnp.int32, (EMBED_DIM, GAUG), 0)
               == jax.lax.broadcasted_iota(jnp.int32, (EMBED_DIM, GAUG), 1)
               ).astype(jnp.bfloat16)
        et_ref[...] = (
            jax.lax.dot_general(eb, eye, (((0,), (0,)), ((), ())),
                                preferred_element_type=jnp.float32)
            + jax.lax.dot_general(e_lo, eye, (((0,), (0,)), ((), ())),
                                  preferred_element_type=jnp.float32))

    x2 = x2_ref[...]
    e2 = e2_acc[...]
    m2 = jax.lax.dot_general((x_blk + x_blk).astype(jnp.bfloat16), eb,
                             (((1,), (0,)), ((), ())),
                             preferred_element_type=jnp.float32)
    dist = (x2 + e2) - m2

    HALF = NUM_EMB // 2
    d0 = dist[:, :HALF]
    d1 = dist[:, HALF:]
    md0 = jnp.min(d0, axis=1, keepdims=True)
    md1 = jnp.min(d1, axis=1, keepdims=True)
    idx0 = jnp.argmin(d0, axis=1, keepdims=True).astype(jnp.int32)
    idx1 = jnp.argmin(d1, axis=1, keepdims=True).astype(jnp.int32)
    u0 = md0.astype(jnp.bfloat16).astype(jnp.float32)
    pick0 = u0 <= md1
    idx = jnp.where(pick0, idx0, idx1 + HALF)
    idx_ref[...] = idx

    lane = jax.lax.broadcasted_iota(jnp.int32, dist.shape, 1)
    oh_b = (lane == idx).astype(jnp.bfloat16)

    xb_aug = jnp.concatenate(
        [x_blk.astype(jnp.bfloat16), jnp.ones((BLOCK, 1), jnp.bfloat16)],
        axis=1)
    un_acc[...] += jax.lax.dot_general(xb_aug, oh_b, (((0,), (0,)), ((), ())),
                                       preferred_element_type=jnp.float32)

    loss_acc[...] += jnp.sum(jnp.where(pick0, md0, md1)).reshape(1, 1)

    @pl.when(i == GRID - 1)
    def _finalize():
        counts = un_acc[EMBED_DIM:, :]
        new_cs = (1 - MOMENTUM) * counts + MOMENTUM * cs_ref[...]
        new_un = (1 - MOMENTUM) * un_acc[:EMBED_DIM, :] + MOMENTUM * un_ref[...]
        n = jnp.sum(new_cs)
        stable_cs = (new_cs + 1e-20) / (n + NUM_EMB * 1e-20) * n
        new_cs_ref[...] = new_cs
        new_un_ref[...] = new_un
        new_e_ref[...] = new_un / stable_cs
        loss_ref[...] = COMMIT * (loss_acc[...] / (N_POINTS * EMBED_DIM))
        ol = counts / N_POINTS
        perp_ref[...] = jnp.exp(-jnp.sum(ol * jnp.log(ol + 1e-20))).reshape(1, 1)


def _sc_gather(et_hbm, idx_hbm, q_hbm, idx_a, idx_b, q_a, q_b, sem):
    c = lax.axis_index("c")
    s = lax.axis_index("s")
    wid = s * NC + c
    base = wid * BPW

    pltpu.sync_copy(idx_hbm.at[pl.ds(base, 128)], idx_a)
    pltpu.sync_copy(idx_hbm.at[pl.ds(base + 128, 128)], idx_b)

    pltpu.async_copy(et_hbm.at[idx_a], q_a, sem).wait()
    pltpu.async_copy(et_hbm.at[idx_b], q_b, sem).wait()
    pltpu.sync_copy(q_a, q_hbm.at[pl.ds(base, 128)])
    pltpu.sync_copy(q_b, q_hbm.at[pl.ds(base + 128, 128)])


def _qst_kernel(q_ref, x_ref, qst_ref):
    x = x_ref[...]
    q = q_ref[...][:, :EMBED_DIM]
    qst_ref[...] = x + (q - x)


@jax.jit
def kernel(x, e, cs, un):
    xf = x.reshape(N_POINTS, EMBED_DIM)
    x2 = jnp.sum(xf * xf, 1, keepdims=True)
    cs2 = cs.reshape(1, NUM_EMB)

    idx, et, loss, perp, new_e, new_cs, new_un = pl.pallas_call(
        _vq_kernel,
        grid=(GRID,),
        in_specs=[
            pl.BlockSpec((BLOCK, EMBED_DIM), lambda i: (i, 0)),
            pl.BlockSpec((BLOCK, 1), lambda i: (i, 0)),
            pl.BlockSpec((EMBED_DIM, NUM_EMB), lambda i: (0, 0)),
            pl.BlockSpec((1, NUM_EMB), lambda i: (0, 0)),
            pl.BlockSpec((EMBED_DIM, NUM_EMB), lambda i: (0, 0)),
        ],
        out_specs=(
            pl.BlockSpec((BLOCK, 1), lambda i: (i, 0)),
            pl.BlockSpec((NUM_EMB, GAUG), lambda i: (0, 0)),
            pl.BlockSpec((1, 1), lambda i: (0, 0)),
            pl.BlockSpec((1, 1), lambda i: (0, 0)),
            pl.BlockSpec((EMBED_DIM, NUM_EMB), lambda i: (0, 0)),
            pl.BlockSpec((1, NUM_EMB), lambda i: (0, 0)),
            pl.BlockSpec((EMBED_DIM, NUM_EMB), lambda i: (0, 0)),
        ),
        out_shape=(
            jax.ShapeDtypeStruct((N_POINTS, 1), jnp.int32),
            jax.ShapeDtypeStruct((NUM_EMB, GAUG), jnp.float32),
            jax.ShapeDtypeStruct((1, 1), jnp.float32),
            jax.ShapeDtypeStruct((1, 1), jnp.float32),
            jax.ShapeDtypeStruct((EMBED_DIM, NUM_EMB), jnp.float32),
            jax.ShapeDtypeStruct((1, NUM_EMB), jnp.float32),
            jax.ShapeDtypeStruct((EMBED_DIM, NUM_EMB), jnp.float32),
        ),
        scratch_shapes=[
            pltpu.VMEM((EMBED_DIM + 1, NUM_EMB), jnp.float32),
            pltpu.VMEM((1, 1), jnp.float32),
            pltpu.VMEM((1, NUM_EMB), jnp.float32),
        ],
    )(xf, x2, e, cs2, un)

    mesh = plsc.VectorSubcoreMesh(core_axis_name="c", subcore_axis_name="s")
    q = pl.kernel(
        _sc_gather,
        out_type=jax.ShapeDtypeStruct((N_POINTS, GAUG), jnp.float32),
        mesh=mesh,
        scratch_types=[
            pltpu.VMEM((128,), jnp.int32),
            pltpu.VMEM((128,), jnp.int32),
            pltpu.VMEM((128, GAUG), jnp.float32),
            pltpu.VMEM((128, GAUG), jnp.float32),
            pltpu.SemaphoreType.DMA,
        ],
    )(et, idx.reshape(N_POINTS))

    qst = pl.pallas_call(
        _qst_kernel,
        out_shape=jax.ShapeDtypeStruct((N_POINTS, EMBED_DIM), jnp.float32),
    )(q, xf)

    return (qst.reshape(x.shape), loss[0, 0], perp[0, 0],
            new_e, new_cs.reshape(NUM_EMB), new_un)


# speedup vs baseline: 2.8505x; 1.0041x over previous
import jax
import jax.numpy as jnp
from jax import lax
from jax.experimental import pallas as pl
from jax.experimental.pallas import tpu as pltpu
from jax.experimental.pallas import tpu_sc as plsc

EMBED_DIM = 32
NUM_EMB = 8192
COMMIT = 0.25
MOMENTUM = 0.9

N_POINTS = 8192
BLOCK = 1024
GRID = N_POINTS // BLOCK
GAUG = 128

NC = 2
NS = 16
NW = NC * NS
BPW = N_POINTS // NW


def _vq_kernel(x_ref, x2_ref, e_ref, cs_ref, un_ref,
               idx_ref, et_ref, loss_ref, perp_ref, new_e_ref, new_cs_ref,
               new_un_ref,
               un_acc, loss_acc, e2_acc):
    i = pl.program_id(0)

    x_blk = x_ref[...]
    e = e_ref[...]
    eb = e.astype(jnp.bfloat16)

    @pl.when(i == 0)
    def _init():
        un_acc[...] = jnp.zeros_like(un_acc)
        loss_acc[...] = jnp.zeros_like(loss_acc)
        e2_acc[...] = jnp.sum(e * e, axis=0, keepdims=True)
        e_lo = (e - eb.astype(jnp.float32)).astype(jnp.bfloat16)
        eye = (jax.lax.broadcasted_iota(jnp.int32, (EMBED_DIM, GAUG), 0)
               == jax.lax.broadcasted_iota(jnp.int32, (EMBED_DIM, GAUG), 1)
               ).astype(jnp.bfloat16)
        et_ref[...] = (
            jax.lax.dot_general(eb, eye, (((0,), (0,)), ((), ())),
                                preferred_element_type=jnp.float32)
            + jax.lax.dot_general(e_lo, eye, (((0,), (0,)), ((), ())),
                                  preferred_element_type=jnp.float32))

    x2 = x2_ref[...]
    e2 = e2_acc[...]
    m2 = jax.lax.dot_general((x_blk + x_blk).astype(jnp.bfloat16), eb,
                             (((1,), (0,)), ((), ())),
                             preferred_element_type=jnp.float32)
    dist = (x2 + e2) - m2

    HALF = NUM_EMB // 2
    d0 = dist[:, :HALF]
    d1 = dist[:, HALF:]
    md0 = jnp.min(d0, axis=1, keepdims=True)
    md1 = jnp.min(d1, axis=1, keepdims=True)
    idx0 = jnp.argmin(d0, axis=1, keepdims=True).astype(jnp.int32)
    idx1 = jnp.argmin(d1, axis=1, keepdims=True).astype(jnp.int32)
    u0 = md0.astype(jnp.bfloat16).astype(jnp.float32)
    pick0 = u0 <= md1
    idx = jnp.where(pick0, idx0, idx1 + HALF)
    idx_ref[...] = idx

    lane = jax.lax.broadcasted_iota(jnp.int32, dist.shape, 1)
    oh_b = (lane == idx).astype(jnp.bfloat16)

    xb_aug = jnp.concatenate(
        [x_blk.astype(jnp.bfloat16), jnp.ones((BLOCK, 1), jnp.bfloat16)],
        axis=1)
    un_acc[...] += jax.lax.dot_general(xb_aug, oh_b, (((0,), (0,)), ((), ())),
                                       preferred_element_type=jnp.float32)

    loss_acc[...] += jnp.sum(jnp.where(pick0, md0, md1)).reshape(1, 1)

    @pl.when(i == GRID - 1)
    def _finalize():
        counts = un_acc[EMBED_DIM:, :]
        new_cs = (1 - MOMENTUM) * counts + MOMENTUM * cs_ref[...]
        new_un = (1 - MOMENTUM) * un_acc[:EMBED_DIM, :] + MOMENTUM * un_ref[...]
        n = jnp.sum(new_cs)
        stable_cs = (new_cs + 1e-20) / (n + NUM_EMB * 1e-20) * n
        new_cs_ref[...] = new_cs
        new_un_ref[...] = new_un
        new_e_ref[...] = new_un / stable_cs
        loss_ref[...] = COMMIT * (loss_acc[...] / (N_POINTS * EMBED_DIM))
        ol = counts / N_POINTS
        perp_ref[...] = jnp.exp(-jnp.sum(ol * jnp.log(ol + 1e-20))).reshape(1, 1)


def _sc_gather(et_hbm, idx_hbm, q_hbm, idx_a, idx_b, q_a, q_b, sem):
    c = lax.axis_index("c")
    s = lax.axis_index("s")
    wid = s * NC + c
    base = wid * BPW

    pltpu.sync_copy(idx_hbm.at[pl.ds(base, 128)], idx_a)
    pltpu.sync_copy(idx_hbm.at[pl.ds(base + 128, 128)], idx_b)

    pltpu.async_copy(et_hbm.at[idx_a], q_a, sem).wait()
    pltpu.async_copy(et_hbm.at[idx_b], q_b, sem).wait()
    pltpu.sync_copy(q_a, q_hbm.at[pl.ds(base, 128)])
    pltpu.sync_copy(q_b, q_hbm.at[pl.ds(base + 128, 128)])


def _qst_kernel(q_ref, x_ref, qst_ref):
    x = x_ref[...]
    q = q_ref[...][:, :EMBED_DIM]
    qst_ref[...] = x + (q - x)


@jax.jit
def kernel(x, e, cs, un):
    xf = x.reshape(N_POINTS, EMBED_DIM)
    x2 = jnp.sum(xf * xf, 1, keepdims=True)
    cs2 = cs.reshape(1, NUM_EMB)

    idx, et, loss, perp, new_e, new_cs, new_un = pl.pallas_call(
        _vq_kernel,
        grid=(GRID,),
        in_specs=[
            pl.BlockSpec((BLOCK, EMBED_DIM), lambda i: (i, 0)),
            pl.BlockSpec((BLOCK, 1), lambda i: (i, 0)),
            pl.BlockSpec((EMBED_DIM, NUM_EMB), lambda i: (0, 0)),
            pl.BlockSpec((1, NUM_EMB), lambda i: (0, 0)),
            pl.BlockSpec((EMBED_DIM, NUM_EMB), lambda i: (0, 0)),
        ],
        out_specs=(
            pl.BlockSpec((BLOCK, 1), lambda i: (i, 0)),
            pl.BlockSpec((NUM_EMB, GAUG), lambda i: (0, 0)),
            pl.BlockSpec((1, 1), lambda i: (0, 0)),
            pl.BlockSpec((1, 1), lambda i: (0, 0)),
            pl.BlockSpec((EMBED_DIM, NUM_EMB), lambda i: (0, 0)),
            pl.BlockSpec((1, NUM_EMB), lambda i: (0, 0)),
            pl.BlockSpec((EMBED_DIM, NUM_EMB), lambda i: (0, 0)),
        ),
        out_shape=(
            jax.ShapeDtypeStruct((N_POINTS, 1), jnp.int32),
            jax.ShapeDtypeStruct((NUM_EMB, GAUG), jnp.float32),
            jax.ShapeDtypeStruct((1, 1), jnp.float32),
            jax.ShapeDtypeStruct((1, 1), jnp.float32),
            jax.ShapeDtypeStruct((EMBED_DIM, NUM_EMB), jnp.float32),
            jax.ShapeDtypeStruct((1, NUM_EMB), jnp.float32),
            jax.ShapeDtypeStruct((EMBED_DIM, NUM_EMB), jnp.float32),
        ),
        scratch_shapes=[
            pltpu.VMEM((EMBED_DIM + 1, NUM_EMB), jnp.float32),
            pltpu.VMEM((1, 1), jnp.float32),
            pltpu.VMEM((1, NUM_EMB), jnp.float32),
        ],
    )(xf, x2, e, cs2, un)

    mesh = plsc.VectorSubcoreMesh(core_axis_name="c", subcore_axis_name="s")
    q = pl.kernel(
        _sc_gather,
        out_type=jax.ShapeDtypeStruct((N_POINTS, GAUG), jnp.float32),
        mesh=mesh,
        scratch_types=[
            pltpu.VMEM((128,), jnp.int32),
            pltpu.VMEM((128,), jnp.int32),
            pltpu.VMEM((128, GAUG), jnp.float32),
            pltpu.VMEM((128, GAUG), jnp.float32),
            pltpu.SemaphoreType.DMA,
        ],
    )(et, idx.reshape(N_POINTS))

    qst = pl.pallas_call(
        _qst_kernel,
        out_shape=jax.ShapeDtypeStruct((N_POINTS, EMBED_DIM), jnp.float32),
    )(q, xf)

    return (qst.reshape(x.shape), loss[0, 0], perp[0, 0],
            new_e, new_cs.reshape(NUM_EMB), new_un)
